# Initial kernel scaffold; baseline (speedup 1.0000x reference)
import jax
import jax.numpy as jnp
from jax.experimental import pallas as pl
from jax.experimental.pallas import tpu as pltpu

B, L, E = 2, 2048, 1024
H, HD = 16, 64
CB, SB, WIN = 16, 16, 64
TOPK = 16
LC = L // CB
NSEL = TOPK * SB
SCALE = 1.0 / 8.0
NEG = -1e30


def _softmax_rows(s):
    m = jnp.max(s, axis=-1, keepdims=True)
    w = jnp.exp(s - m)
    return w / jnp.sum(w, axis=-1, keepdims=True)


def _dotT(a, b):
    return jax.lax.dot_general(a, b, (((1,), (1,)), ((), ())),
                               preferred_element_type=jnp.float32)


def _nsa_kernel(x_ref, w3_ref, b3_ref, w1r_ref, bc1_ref, w2t_ref, bc2_ref,
                wg_ref, bg_ref, out_ref):
    xb = x_ref[0]
    qkv = jnp.dot(xb, w3_ref[0],
                  preferred_element_type=jnp.float32) + b3_ref[0]
    qb = qkv[:, :HD]
    kb = qkv[:, HD:2 * HD]
    vb = qkv[:, 2 * HD:]

    kb3 = kb.reshape(LC, CB, HD)
    vb3 = vb.reshape(LC, CB, HD)
    kjs = [kb3[:, j, :] for j in range(CB)]
    vjs = [vb3[:, j, :] for j in range(CB)]

    h1k = bc1_ref[:]
    h1v = bc1_ref[:]
    for j in range(CB):
        w1j = w1r_ref[j]
        h1k = h1k + jnp.dot(kjs[j], w1j, preferred_element_type=jnp.float32)
        h1v = h1v + jnp.dot(vjs[j], w1j, preferred_element_type=jnp.float32)
    kc = jnp.dot(jnp.maximum(h1k, 0.0), w2t_ref[:],
                 preferred_element_type=jnp.float32) + bc2_ref[:]
    vc = jnp.dot(jnp.maximum(h1v, 0.0), w2t_ref[:],
                 preferred_element_type=jnp.float32) + bc2_ref[:]

    wn = _softmax_rows(_dotT(qb, kc) * SCALE)
    attn_c = jnp.dot(wn, vc, preferred_element_type=jnp.float32)
    bs = jnp.sum(wn, axis=0, keepdims=True)

    iota = jax.lax.broadcasted_iota(jnp.int32, (1, LC), 1)
    ohs = []
    for _ in range(TOPK):
        m = jnp.max(bs, axis=-1, keepdims=True)
        fi = jnp.min(jnp.where(bs >= m, iota, LC), axis=-1, keepdims=True)
        oh = iota == fi
        ohs.append(oh.astype(jnp.float32))
        bs = jnp.where(oh, NEG, bs)
    sel = jnp.concatenate(ohs, axis=0)

    ksel = jnp.concatenate(
        [jnp.dot(sel, kjs[j], preferred_element_type=jnp.float32)
         for j in range(CB)], axis=0)
    vsel = jnp.concatenate(
        [jnp.dot(sel, vjs[j], preferred_element_type=jnp.float32)
         for j in range(CB)], axis=0)

    ws = _softmax_rows(_dotT(qb, ksel) * SCALE)
    attn_s = jnp.dot(ws, vsel, preferred_element_type=jnp.float32)

    wwin = _softmax_rows(_dotT(qb, kb[L - WIN:, :]) * SCALE)
    attn_w = jnp.dot(wwin, vb[L - WIN:, :], preferred_element_type=jnp.float32)

    g = _softmax_rows(jnp.dot(qb, wg_ref[:],
                              preferred_element_type=jnp.float32) + bg_ref[:])
    out = (g[:, 0:1] * attn_c + g[:, 1:2] * attn_s + g[:, 2:3] * attn_w)
    out_ref[0, :, 0, :] = out


def kernel(x, Wq, bq, Wk, bk, Wv, bv, Wc1, bc1, Wc2, bc2, Wg, bg):
    f32 = jnp.float32
    WqT = Wq.T.reshape(E, H, HD)
    WkT = Wk.T.reshape(E, H, HD)
    WvT = Wv.T.reshape(E, H, HD)
    W3 = jnp.concatenate([WqT, WkT, WvT], axis=-1).transpose(1, 0, 2)
    b3 = jnp.concatenate([bq.reshape(H, 1, HD), bk.reshape(H, 1, HD),
                          bv.reshape(H, 1, HD)], axis=-1)
    W1r = Wc1.T.reshape(CB, HD, HD // 2)
    bc1r = bc1.reshape(1, HD // 2)
    W2T = Wc2.T
    bc2r = bc2.reshape(1, HD)
    Wgp = jnp.zeros((HD, 128), f32).at[:, :3].set(Wg.T)
    bgp = jnp.full((1, 128), NEG, f32).at[0, :3].set(bg)

    out = pl.pallas_call(
        _nsa_kernel,
        grid=(B, H),
        in_specs=[
            pl.BlockSpec((1, L, E), lambda b, h: (b, 0, 0)),
            pl.BlockSpec((1, E, 3 * HD), lambda b, h: (h, 0, 0)),
            pl.BlockSpec((1, 1, 3 * HD), lambda b, h: (h, 0, 0)),
            pl.BlockSpec((CB, HD, HD // 2), lambda b, h: (0, 0, 0)),
            pl.BlockSpec((1, HD // 2), lambda b, h: (0, 0)),
            pl.BlockSpec((HD // 2, HD), lambda b, h: (0, 0)),
            pl.BlockSpec((1, HD), lambda b, h: (0, 0)),
            pl.BlockSpec((HD, 128), lambda b, h: (0, 0)),
            pl.BlockSpec((1, 128), lambda b, h: (0, 0)),
        ],
        out_specs=pl.BlockSpec((1, L, 1, HD), lambda b, h: (b, 0, h, 0)),
        out_shape=jax.ShapeDtypeStruct((B, L, H, HD), jnp.float32),
        compiler_params=pltpu.CompilerParams(
            dimension_semantics=("parallel", "arbitrary")),
    )(x, W3, b3, W1r, bc1r, W2T, bc2r, Wgp, bgp)
    return out.reshape(B, L, E)


# speedup vs baseline: 1.0957x; 1.0957x over previous
import jax
import jax.numpy as jnp
from jax.experimental import pallas as pl
from jax.experimental.pallas import tpu as pltpu

B, L, E = 2, 2048, 1024
H, HD = 16, 64
CB, SB, WIN = 16, 16, 64
TOPK = 16
LC = L // CB
NSEL = TOPK * SB
SCALE = 1.0 / 8.0
NEG = -1e30
HP = H // 2


def _softmax_rows(s):
    m = jnp.max(s, axis=-1, keepdims=True)
    w = jnp.exp(s - m)
    return w / jnp.sum(w, axis=-1, keepdims=True)


def _dotT(a, b):
    return jax.lax.dot_general(a, b, (((1,), (1,)), ((), ())),
                               preferred_element_type=jnp.float32)


def _one_head(qb, kb, vb, w1r_ref, bc1_ref, w2t_ref, bc2_ref, wg_ref, bg_ref):
    kb3 = kb.reshape(LC, CB, HD)
    vb3 = vb.reshape(LC, CB, HD)
    kjs = [kb3[:, j, :] for j in range(CB)]
    vjs = [vb3[:, j, :] for j in range(CB)]

    h1k = bc1_ref[:]
    h1v = bc1_ref[:]
    for j in range(CB):
        w1j = w1r_ref[j]
        h1k = h1k + jnp.dot(kjs[j], w1j, preferred_element_type=jnp.float32)
        h1v = h1v + jnp.dot(vjs[j], w1j, preferred_element_type=jnp.float32)
    kc = jnp.dot(jnp.maximum(h1k, 0.0), w2t_ref[:],
                 preferred_element_type=jnp.float32) + bc2_ref[:]
    vc = jnp.dot(jnp.maximum(h1v, 0.0), w2t_ref[:],
                 preferred_element_type=jnp.float32) + bc2_ref[:]

    wn = _softmax_rows(_dotT(qb, kc) * SCALE)
    attn_c = jnp.dot(wn, vc, preferred_element_type=jnp.float32)
    bs = jnp.sum(wn, axis=0, keepdims=True)

    iota = jax.lax.broadcasted_iota(jnp.int32, (1, LC), 1)
    ohs = []
    for _ in range(TOPK):
        m = jnp.max(bs, axis=-1, keepdims=True)
        fi = jnp.min(jnp.where(bs >= m, iota, LC), axis=-1, keepdims=True)
        oh = iota == fi
        ohs.append(oh.astype(jnp.float32))
        bs = jnp.where(oh, NEG, bs)
    sel = jnp.concatenate(ohs, axis=0)

    ksel = jnp.concatenate(
        [jnp.dot(sel, kjs[j], preferred_element_type=jnp.float32)
         for j in range(CB)], axis=0)
    vsel = jnp.concatenate(
        [jnp.dot(sel, vjs[j], preferred_element_type=jnp.float32)
         for j in range(CB)], axis=0)

    ws = _softmax_rows(_dotT(qb, ksel) * SCALE)
    attn_s = jnp.dot(ws, vsel, preferred_element_type=jnp.float32)

    wwin = _softmax_rows(_dotT(qb, kb[L - WIN:, :]) * SCALE)
    attn_w = jnp.dot(wwin, vb[L - WIN:, :], preferred_element_type=jnp.float32)

    g = _softmax_rows(jnp.dot(qb, wg_ref[:],
                              preferred_element_type=jnp.float32) + bg_ref[:])
    return g[:, 0:1] * attn_c + g[:, 1:2] * attn_s + g[:, 2:3] * attn_w


def _nsa_kernel(x_ref, w3_ref, b3_ref, w1r_ref, bc1_ref, w2t_ref, bc2_ref,
                wg_ref, bg_ref, out_ref):
    xb = x_ref[0]
    qkv = jnp.dot(xb, w3_ref[:],
                  preferred_element_type=jnp.float32) + b3_ref[:]
    outs = []
    for i in range(2):
        o = i * 3 * HD
        qb = qkv[:, o:o + HD]
        kb = qkv[:, o + HD:o + 2 * HD]
        vb = qkv[:, o + 2 * HD:o + 3 * HD]
        outs.append(_one_head(qb, kb, vb, w1r_ref, bc1_ref, w2t_ref,
                              bc2_ref, wg_ref, bg_ref))
    out_ref[0] = jnp.concatenate(outs, axis=1)


def kernel(x, Wq, bq, Wk, bk, Wv, bv, Wc1, bc1, Wc2, bc2, Wg, bg):
    f32 = jnp.float32
    WqT = Wq.T.reshape(E, H, HD)
    WkT = Wk.T.reshape(E, H, HD)
    WvT = Wv.T.reshape(E, H, HD)
    W3 = jnp.concatenate([WqT, WkT, WvT], axis=-1).reshape(E, H * 3 * HD)
    b3 = jnp.concatenate([bq.reshape(H, HD), bk.reshape(H, HD),
                          bv.reshape(H, HD)], axis=-1).reshape(1, H * 3 * HD)
    W1r = Wc1.T.reshape(CB, HD, HD // 2)
    bc1r = bc1.reshape(1, HD // 2)
    W2T = Wc2.T
    bc2r = bc2.reshape(1, HD)
    Wgp = jnp.zeros((HD, 128), f32).at[:, :3].set(Wg.T)
    bgp = jnp.full((1, 128), NEG, f32).at[0, :3].set(bg)

    out = pl.pallas_call(
        _nsa_kernel,
        grid=(B, HP),
        in_specs=[
            pl.BlockSpec((1, L, E), lambda b, g: (b, 0, 0)),
            pl.BlockSpec((E, 6 * HD), lambda b, g: (0, g)),
            pl.BlockSpec((1, 6 * HD), lambda b, g: (0, g)),
            pl.BlockSpec((CB, HD, HD // 2), lambda b, g: (0, 0, 0)),
            pl.BlockSpec((1, HD // 2), lambda b, g: (0, 0)),
            pl.BlockSpec((HD // 2, HD), lambda b, g: (0, 0)),
            pl.BlockSpec((1, HD), lambda b, g: (0, 0)),
            pl.BlockSpec((HD, 128), lambda b, g: (0, 0)),
            pl.BlockSpec((1, 128), lambda b, g: (0, 0)),
        ],
        out_specs=pl.BlockSpec((1, L, 128), lambda b, g: (b, 0, g)),
        out_shape=jax.ShapeDtypeStruct((B, L, E), jnp.float32),
        compiler_params=pltpu.CompilerParams(
            dimension_semantics=("parallel", "arbitrary")),
    )(x, W3, b3, W1r, bc1r, W2T, bc2r, Wgp, bgp)
    return out
